# baseline (speedup 1.0000x reference)
import jax
import jax.numpy as jnp
from jax import lax
from jax.experimental import pallas as pl
from jax.experimental.pallas import tpu as pltpu
from jax.experimental.pallas import tpu_sc as plsc

import functools

BATCH = 16384
HIST = 50
EMBED_DIM = 64

NUM_CORES = 2
NUM_SUBCORES = 16
NW = NUM_CORES * NUM_SUBCORES
CHUNK = 128
B_TOTAL = BATCH * HIST
BLOCKS = B_TOTAL // (NW * CHUNK)
BPS = 2
NSUPER = BLOCKS // BPS
FT = EMBED_DIM // 8
BT = BATCH // CHUNK

_mesh = plsc.VectorSubcoreMesh(core_axis_name="c", subcore_axis_name="s")


@functools.partial(
    pl.kernel,
    mesh=_mesh,
    compiler_params=pltpu.CompilerParams(
        use_tc_tiling_on_sc=False, needs_layout_passes=False
    ),
    out_type=jax.ShapeDtypeStruct((HIST, FT, BT, 8 * CHUNK), jnp.float32),
    scratch_types=[
        pltpu.VMEM((BLOCKS, CHUNK), jnp.int32),
        pltpu.VMEM((2, BPS * CHUNK, EMBED_DIM), jnp.float32),
        pltpu.VMEM((2, BPS, FT, 8 * CHUNK), jnp.float32),
        [pltpu.SemaphoreType.DMA] * 2,
        [pltpu.SemaphoreType.DMA] * 2,
    ],
)
def _gather_kernel(table_hbm, idx_hbm, out_hbm, idx_v, rows_v, trans_v, gsems, wsems):
    wid = lax.axis_index("s") * NUM_CORES + lax.axis_index("c")
    pltpu.sync_copy(idx_hbm.at[wid], idx_v)

    lanes = lax.iota(jnp.int32, 16)
    perm = [(lanes + d) & 15 for d in range(16)]

    def start_bank(t, bank):
        for u in range(BPS):
            pltpu.async_copy(
                table_hbm.at[idx_v.at[BPS * t + u]],
                rows_v.at[bank, pl.ds(u * CHUNK, CHUNK)],
                gsems[bank],
            )

    def wait_gathers(bank):
        pltpu.make_async_copy(
            table_hbm.at[idx_v.at[0]], rows_v.at[bank], gsems[bank]
        ).wait()

    def wait_writes(bank):
        for blk in range(BPS):
            pltpu.make_async_copy(
                out_hbm.at[0, pl.ds(0, FT), 0], trans_v.at[bank, blk], wsems[bank]
            ).wait()

    start_bank(0, 0)

    def group(g, _):
        for b in range(2):
            t = 2 * g + b

            @pl.when(t + 1 < NSUPER)
            def _():
                start_bank(t + 1, 1 - b)

            wait_gathers(b)

            @pl.when(t >= 2)
            def _():
                wait_writes(b)

            for blk in range(BPS):
                @plsc.parallel_loop(0, 32, unroll=8)
                def _(i, _blk=blk, _b=b):
                    f0 = (i // 8) * 16
                    c0 = (i % 8) * 16
                    ridx = _blk * CHUNK + c0 + lanes
                    for d in range(16):
                        fvec = f0 + perm[d]
                        val = plsc.load_gather(rows_v.at[_b], [ridx, fvec])
                        fr_vec = fvec // 8
                        rc_vec = (fvec & 7) * CHUNK + c0 + lanes
                        plsc.store_scatter(
                            trans_v.at[_b, _blk], [fr_vec, rc_vec], val
                        )

                f = wid * BLOCKS + BPS * t + blk
                h = f // BT
                bt = f % BT
                pltpu.async_copy(
                    trans_v.at[b, blk], out_hbm.at[h, pl.ds(0, FT), bt], wsems[b]
                )
        return ()

    lax.fori_loop(0, NSUPER // 2, group, ())
    wait_writes(0)
    wait_writes(1)


def kernel(x, table):
    xf = x.T.reshape(NW, BLOCKS, CHUNK).astype(jnp.int32)
    out4 = _gather_kernel(table, xf)
    out5 = out4.reshape(HIST, FT, BT, 8, CHUNK)
    return out5.transpose(2, 4, 0, 1, 3).reshape(BATCH, HIST, EMBED_DIM)


# speedup vs baseline: 1.3216x; 1.3216x over previous
import jax
import jax.numpy as jnp
from jax import lax
from jax.experimental import pallas as pl
from jax.experimental.pallas import tpu as pltpu
from jax.experimental.pallas import tpu_sc as plsc

import functools

BATCH = 16384
HIST = 50
EMBED_DIM = 64

NUM_CORES = 2
NUM_SUBCORES = 16
NW = NUM_CORES * NUM_SUBCORES
CHUNK = 128
B_TOTAL = BATCH * HIST
BLOCKS = B_TOTAL // (NW * CHUNK)
BPS = 2
NSUPER = BLOCKS // BPS
FT = EMBED_DIM // 8
BT = BATCH // CHUNK

_mesh = plsc.VectorSubcoreMesh(core_axis_name="c", subcore_axis_name="s")


@functools.partial(
    pl.kernel,
    mesh=_mesh,
    compiler_params=pltpu.CompilerParams(
        use_tc_tiling_on_sc=False, needs_layout_passes=False
    ),
    out_type=jax.ShapeDtypeStruct((HIST, FT, BT, 8 * CHUNK), jnp.float32),
    scratch_types=[
        pltpu.VMEM((BLOCKS, CHUNK), jnp.int32),
        pltpu.VMEM((2, BPS * CHUNK, EMBED_DIM), jnp.float32),
        pltpu.VMEM((2, FT, BPS, 8 * CHUNK), jnp.float32),
        [pltpu.SemaphoreType.DMA] * 2,
        [pltpu.SemaphoreType.DMA] * 2,
    ],
)
def _gather_kernel(table_hbm, idx_hbm, out_hbm, idx_v, rows_v, trans_v, gsems, wsems):
    wid = lax.axis_index("s") * NUM_CORES + lax.axis_index("c")
    pltpu.sync_copy(idx_hbm.at[wid], idx_v)

    lanes = lax.iota(jnp.int32, 16)
    perm = [(lanes + d) & 15 for d in range(16)]

    def start_bank(t, bank):
        for u in range(BPS):
            pltpu.async_copy(
                table_hbm.at[idx_v.at[BPS * t + u]],
                rows_v.at[bank, pl.ds(u * CHUNK, CHUNK)],
                gsems[bank],
            )

    def wait_gathers(bank):
        pltpu.make_async_copy(
            table_hbm.at[idx_v.at[0]], rows_v.at[bank], gsems[bank]
        ).wait()

    def wait_writes(bank):
        pltpu.make_async_copy(
            out_hbm.at[0, pl.ds(0, FT), pl.ds(0, BPS)],
            trans_v.at[bank],
            wsems[bank],
        ).wait()

    start_bank(0, 0)

    def group(g, _):
        for b in range(2):
            t = 2 * g + b

            @pl.when(t + 1 < NSUPER)
            def _():
                start_bank(t + 1, 1 - b)

            wait_gathers(b)

            @pl.when(t >= 2)
            def _():
                wait_writes(b)

            for blk in range(BPS):
                blkvec = jnp.full((16,), blk, jnp.int32)

                @plsc.parallel_loop(0, 32, unroll=4)
                def _(i, _blk=blk, _b=b, _blkvec=blkvec):
                    f0 = (i // 8) * 16
                    c0 = (i % 8) * 16
                    ridx = _blk * CHUNK + c0 + lanes
                    for d in range(16):
                        fvec = f0 + perm[d]
                        val = plsc.load_gather(rows_v.at[_b], [ridx, fvec])
                        fr_vec = fvec // 8
                        rc_vec = (fvec & 7) * CHUNK + c0 + lanes
                        plsc.store_scatter(
                            trans_v.at[_b], [fr_vec, _blkvec, rc_vec], val
                        )

            f = wid * BLOCKS + BPS * t
            h = f // BT
            bt = f % BT
            pltpu.async_copy(
                trans_v.at[b],
                out_hbm.at[h, pl.ds(0, FT), pl.ds(bt, BPS)],
                wsems[b],
            )
        return ()

    lax.fori_loop(0, NSUPER // 2, group, ())
    wait_writes(0)
    wait_writes(1)


def kernel(x, table):
    xf = x.T.reshape(NW, BLOCKS, CHUNK).astype(jnp.int32)
    out4 = _gather_kernel(table, xf)
    out5 = out4.reshape(HIST, FT, BT, 8, CHUNK)
    return out5.transpose(2, 4, 0, 1, 3).reshape(BATCH, HIST, EMBED_DIM)
